# baseline (speedup 1.0000x reference)
import functools

import jax
import jax.numpy as jnp
from jax import lax
from jax.experimental import pallas as pl
from jax.experimental.pallas import tpu as pltpu
from jax.experimental.pallas import tpu_sc as plsc

DIM = 256
KNN = 16
XPAD = 128
TW = 2 * DIM + XPAD


def _knn_body(xyzT_ref, xyzb_ref, idx_ref, *, blkr, n):
    nv = n // 128
    f32 = jnp.float32
    i32 = jnp.int32
    BIGF = f32(1.0e38)
    BIGI = i32(0x7F000000)
    xb = xyzb_ref[0]
    xc = [xb[:, c:c + 1] for c in range(3)]
    iota_l = jax.lax.broadcasted_iota(i32, (blkr, 128), 1)

    def key_slice(v):
        acc = jnp.zeros((blkr, 128), f32)
        for c in range(3):
            yc = xyzT_ref[0, c:c + 1, v * 128:(v + 1) * 128]
            d = xc[c] - yc
            acc = acc + d * d
        acc = jnp.maximum(acc, f32(1.0e-37))
        bits = (jax.lax.bitcast_convert_type(acc, i32) & i32(~31)) | i32(v)
        return jax.lax.bitcast_convert_type(bits, f32)

    M = [jnp.full((blkr, 128), BIGF, f32) for _ in range(4)]
    for v in range(nv):
        x = key_slice(v)
        for lvl in range(4):
            lo = jnp.minimum(M[lvl], x)
            x = jnp.maximum(M[lvl], x)
            M[lvl] = lo

    M1, M2, M3, M4 = M
    outs = []
    for _ in range(KNN):
        m = jnp.min(M1, axis=1, keepdims=True)
        eq = M1 == m
        lmin = jnp.min(jnp.where(eq, iota_l, i32(128)), axis=1, keepdims=True)
        pop = eq & (iota_l == lmin)
        fi = ((jax.lax.bitcast_convert_type(M1, i32) & i32(31)) * i32(128)
              + iota_l)
        a = jnp.min(jnp.where(pop, fi, BIGI), axis=1, keepdims=True)
        outs.append(a)
        M1 = jnp.where(pop, M2, M1)
        M2 = jnp.where(pop, M3, M2)
        M3 = jnp.where(pop, M4, M3)
        M4 = jnp.where(pop, BIGF, M4)
    fast = jnp.concatenate(outs, axis=1)
    bad = jnp.any(M1 >= f32(1.0e37))

    def slow():
        kk = jnp.concatenate([key_slice(v) for v in range(nv)], axis=1)
        iota_n = jax.lax.broadcasted_iota(i32, (blkr, n), 1)
        fi2 = ((jax.lax.bitcast_convert_type(kk, i32) & i32(31)) * i32(128)
               + (iota_n & i32(127)))
        res = []
        for _ in range(KNN):
            mm = jnp.min(kk, axis=1, keepdims=True)
            aa = jnp.min(jnp.where(kk == mm, fi2, BIGI), axis=1,
                         keepdims=True)
            res.append(aa)
            kk = jnp.where(fi2 == aa, BIGF, kk)
        return jnp.concatenate(res, axis=1)

    idx_ref[0] = jax.lax.cond(bad, slow, lambda: fast)


def _make_sc_gather(nrows):
    nw = 32
    ch = 64
    rows_w = nrows // nw
    nch = rows_w // ch
    mesh = plsc.VectorSubcoreMesh(core_axis_name="c", subcore_axis_name="s")
    f32 = jnp.float32

    @functools.partial(
        pl.kernel, mesh=mesh,
        out_type=jax.ShapeDtypeStruct((nrows, TW), f32),
        scratch_types=[
            pltpu.VMEM((nch, ch), jnp.int32),
            pltpu.VMEM((ch, TW), f32),
            pltpu.VMEM((ch, TW), f32),
            pltpu.SemaphoreType.DMA,
            pltpu.SemaphoreType.DMA,
        ],
    )
    def gather(tab_hbm, idx_hbm, out_hbm, idx_v, buf0, buf1, s0, s1):
        wid = lax.axis_index("s") * 2 + lax.axis_index("c")
        base = wid * rows_w
        pltpu.sync_copy(idx_hbm.at[wid], idx_v)
        pltpu.async_copy(tab_hbm.at[idx_v.at[0]], buf0, s0)
        pltpu.async_copy(tab_hbm.at[idx_v.at[1]], buf1, s1)

        def body(gg, _):
            g0 = gg * 2
            pltpu.make_async_copy(tab_hbm.at[idx_v.at[g0]], buf0, s0).wait()
            pltpu.sync_copy(buf0, out_hbm.at[pl.ds(base + g0 * ch, ch)])

            @pl.when(g0 + 2 < nch)
            def _():
                pltpu.async_copy(tab_hbm.at[idx_v.at[g0 + 2]], buf0, s0)

            pltpu.make_async_copy(tab_hbm.at[idx_v.at[g0 + 1]], buf1,
                                  s1).wait()
            pltpu.sync_copy(buf1,
                            out_hbm.at[pl.ds(base + (g0 + 1) * ch, ch)])

            @pl.when(g0 + 3 < nch)
            def _():
                pltpu.async_copy(tab_hbm.at[idx_v.at[g0 + 3]], buf1, s1)

            return 0

        lax.fori_loop(0, nch // 2, body, 0)

    return gather


def _pointwise_body(feat_ref, xp_ref, makT_ref, wvT_ref, tab_ref):
    f = feat_ref[...]
    ak = jnp.dot(f, makT_ref[...], preferred_element_type=jnp.float32)
    vf = jnp.dot(f, wvT_ref[...], preferred_element_type=jnp.float32)
    tab_ref[...] = jnp.concatenate([ak, vf, xp_ref[...]], axis=1)


def _attn_body(feat_ref, tabg_ref, xq_ref, maqT_ref, p1wT_ref,
               m1T_ref, a2wT_ref, p2wT_ref, bias_ref, out_ref, *, blk):
    r = blk * KNN
    p1b = bias_ref[0:1, :]
    c1 = bias_ref[1:2, :]
    p2b = bias_ref[2:3, :]
    gamma = bias_ref[3:4, :]
    beta = bias_ref[4:5, :]

    tabg = tabg_ref[...]
    akg = tabg[:, :DIM]
    vfg = tabg[:, DIM:2 * DIM]
    xg = tabg[:, 2 * DIM:]
    rel = (xq_ref[...][:, None, :]
           - xg.reshape(blk, KNN, XPAD)).reshape(r, XPAD)
    u = jnp.maximum(
        jnp.dot(rel, p1wT_ref[...], preferred_element_type=jnp.float32) + p1b,
        0.0)
    t = jnp.dot(u, m1T_ref[...], preferred_element_type=jnp.float32)
    f = feat_ref[...]
    aqc = jnp.dot(f, maqT_ref[...], preferred_element_type=jnp.float32) + c1
    h = (t.reshape(blk, KNN, DIM) - akg.reshape(blk, KNN, DIM)
         + aqc[:, None, :])
    h2 = jnp.maximum(h, 0.0)
    h3 = jnp.dot(h2.reshape(r, DIM), a2wT_ref[...],
                 preferred_element_type=jnp.float32).reshape(blk, KNN, DIM)
    m = jnp.max(h3, axis=1, keepdims=True)
    e = jnp.exp(h3 - m)
    s = jnp.sum(e, axis=1, keepdims=True)
    attn = e / s
    out_v = jnp.sum(vfg.reshape(blk, KNN, DIM) * attn, axis=1)
    su = jnp.sum(u.reshape(blk, KNN, DIM) * attn, axis=1)
    y = (out_v + jnp.dot(su, p2wT_ref[...], preferred_element_type=jnp.float32)
         + p2b + f)
    mu = jnp.mean(y, axis=1, keepdims=True)
    yc = y - mu
    var = jnp.mean(yc * yc, axis=1, keepdims=True)
    out_ref[...] = yc * jax.lax.rsqrt(var + 1e-5) * gamma + beta


def kernel(xyz, feat, Wq, Wk, Wv, p1w, p1b, p2w, p2b, a1w, a1b, a2w, a2b,
           gamma, beta):
    B, N, _ = xyz.shape
    BN = B * N
    f32 = jnp.float32

    maqT = (a1w @ Wq).T
    makT = (a1w @ Wk).T
    wvT = Wv.T
    m1T = (a1w @ p2w).T
    a2wT = a2w.T
    p2wT = p2w.T
    p1wT = jnp.zeros((XPAD, DIM), f32).at[:3, :].set(p1w.T)
    c1 = a1b + a1w @ p2b
    bias_tab = jnp.zeros((8, DIM), f32)
    bias_tab = bias_tab.at[0].set(p1b).at[1].set(c1).at[2].set(p2b)
    bias_tab = bias_tab.at[3].set(gamma).at[4].set(beta)

    blkr = 64
    xyzT = jnp.zeros((B, 8, N), f32).at[:, :3, :].set(
        jnp.transpose(xyz, (0, 2, 1)))
    xb8 = jnp.zeros((B, N, 8), f32).at[..., :3].set(xyz)
    knn_idx = pl.pallas_call(
        functools.partial(_knn_body, blkr=blkr, n=N),
        grid=(B, N // blkr),
        in_specs=[
            pl.BlockSpec((1, 8, N), lambda b, i: (b, 0, 0)),
            pl.BlockSpec((1, blkr, 8), lambda b, i: (b, i, 0)),
        ],
        out_specs=pl.BlockSpec((1, blkr, KNN), lambda b, i: (b, i, 0)),
        out_shape=jax.ShapeDtypeStruct((B, N, KNN), jnp.int32),
    )(xyzT, xb8)

    feat2 = feat.reshape(BN, DIM)
    xyzP = jnp.zeros((BN, XPAD), f32).at[:, :3].set(xyz.reshape(BN, 3))
    blk_a = 512
    tab = pl.pallas_call(
        _pointwise_body,
        grid=(BN // blk_a,),
        in_specs=[
            pl.BlockSpec((blk_a, DIM), lambda i: (i, 0)),
            pl.BlockSpec((blk_a, XPAD), lambda i: (i, 0)),
            pl.BlockSpec((DIM, DIM), lambda i: (0, 0)),
            pl.BlockSpec((DIM, DIM), lambda i: (0, 0)),
        ],
        out_specs=pl.BlockSpec((blk_a, TW), lambda i: (i, 0)),
        out_shape=jax.ShapeDtypeStruct((BN, TW), f32),
    )(feat2, xyzP, makT, wvT)

    nrows = BN * KNN
    gidx = (knn_idx + (jnp.arange(B, dtype=jnp.int32) * N)[:, None, None])
    gidx3 = gidx.reshape(32, nrows // (32 * 64), 64)
    tabg = _make_sc_gather(nrows)(tab, gidx3)

    blk = 64
    rblk = blk * KNN
    wspec = pl.BlockSpec((DIM, DIM), lambda i: (0, 0))
    y = pl.pallas_call(
        functools.partial(_attn_body, blk=blk),
        grid=(BN // blk,),
        in_specs=[
            pl.BlockSpec((blk, DIM), lambda i: (i, 0)),
            pl.BlockSpec((rblk, TW), lambda i: (i, 0)),
            pl.BlockSpec((blk, XPAD), lambda i: (i, 0)),
            wspec,
            pl.BlockSpec((XPAD, DIM), lambda i: (0, 0)),
            wspec,
            wspec,
            wspec,
            pl.BlockSpec((8, DIM), lambda i: (0, 0)),
        ],
        out_specs=pl.BlockSpec((blk, DIM), lambda i: (i, 0)),
        out_shape=jax.ShapeDtypeStruct((BN, DIM), f32),
    )(feat2, tabg, xyzP, maqT, p1wT, m1T, a2wT, p2wT, bias_tab)

    return y.reshape(B, N, DIM)


# speedup vs baseline: 1.0867x; 1.0867x over previous
import functools

import jax
import jax.numpy as jnp
from jax import lax
from jax.experimental import pallas as pl
from jax.experimental.pallas import tpu as pltpu
from jax.experimental.pallas import tpu_sc as plsc

DIM = 256
KNN = 16
XPAD = 128
TW = 2 * DIM + XPAD


def _knn_body(xyzT_ref, xyzb_ref, idx_ref, *, blkr, n):
    nv = n // 128
    f32 = jnp.float32
    i32 = jnp.int32
    BIGF = f32(1.0e38)
    BIGI = i32(0x7F000000)
    xb = xyzb_ref[0]
    xc = [xb[:, c:c + 1] for c in range(3)]
    iota_l = jax.lax.broadcasted_iota(i32, (blkr, 128), 1)

    def key_slice(v):
        acc = jnp.zeros((blkr, 128), f32)
        for c in range(3):
            yc = xyzT_ref[0, c:c + 1, v * 128:(v + 1) * 128]
            d = xc[c] - yc
            acc = acc + d * d
        acc = jnp.maximum(acc, f32(1.0e-37))
        bits = (jax.lax.bitcast_convert_type(acc, i32) & i32(~31)) | i32(v)
        return jax.lax.bitcast_convert_type(bits, f32)

    M = [jnp.full((blkr, 128), BIGF, f32) for _ in range(4)]
    for v in range(nv):
        x = key_slice(v)
        for lvl in range(4):
            lo = jnp.minimum(M[lvl], x)
            x = jnp.maximum(M[lvl], x)
            M[lvl] = lo

    M1, M2, M3, M4 = M
    outs = []
    for _ in range(KNN):
        m = jnp.min(M1, axis=1, keepdims=True)
        eq = M1 == m
        lmin = jnp.min(jnp.where(eq, iota_l, i32(128)), axis=1, keepdims=True)
        pop = eq & (iota_l == lmin)
        fi = ((jax.lax.bitcast_convert_type(M1, i32) & i32(31)) * i32(128)
              + iota_l)
        a = jnp.min(jnp.where(pop, fi, BIGI), axis=1, keepdims=True)
        outs.append(a)
        M1 = jnp.where(pop, M2, M1)
        M2 = jnp.where(pop, M3, M2)
        M3 = jnp.where(pop, M4, M3)
        M4 = jnp.where(pop, BIGF, M4)
    fast = jnp.concatenate(outs, axis=1)
    bad = jnp.any(M1 >= f32(1.0e37))

    def slow():
        kk = jnp.concatenate([key_slice(v) for v in range(nv)], axis=1)
        iota_n = jax.lax.broadcasted_iota(i32, (blkr, n), 1)
        fi2 = ((jax.lax.bitcast_convert_type(kk, i32) & i32(31)) * i32(128)
               + (iota_n & i32(127)))
        res = []
        for _ in range(KNN):
            mm = jnp.min(kk, axis=1, keepdims=True)
            aa = jnp.min(jnp.where(kk == mm, fi2, BIGI), axis=1,
                         keepdims=True)
            res.append(aa)
            kk = jnp.where(fi2 == aa, BIGF, kk)
        return jnp.concatenate(res, axis=1)

    idx_ref[0] = jax.lax.cond(bad, slow, lambda: fast)


def _make_sc_gather(nrows):
    nw = 32
    ch = 64
    rows_w = nrows // nw
    nch = rows_w // ch
    mesh = plsc.VectorSubcoreMesh(core_axis_name="c", subcore_axis_name="s")
    f32 = jnp.float32
    bf16 = jnp.bfloat16

    @functools.partial(
        pl.kernel, mesh=mesh,
        out_type=[
            jax.ShapeDtypeStruct((nrows, DIM), jnp.int32),
            jax.ShapeDtypeStruct((nrows, XPAD), f32),
        ],
        scratch_types=[
            pltpu.VMEM((nch, ch), jnp.int32),
            pltpu.VMEM((ch, DIM), jnp.int32),
            pltpu.VMEM((ch, DIM), jnp.int32),
            pltpu.VMEM((ch, XPAD), f32),
            pltpu.VMEM((ch, XPAD), f32),
            pltpu.SemaphoreType.DMA,
            pltpu.SemaphoreType.DMA,
        ],
    )
    def gather(tab_hbm, xyz_hbm, idx_hbm, tabg_hbm, xg_hbm,
               idx_v, buf0, buf1, xb0, xb1, s0, s1):
        wid = lax.axis_index("s") * 2 + lax.axis_index("c")
        base = wid * rows_w
        pltpu.sync_copy(idx_hbm.at[wid], idx_v)
        pltpu.async_copy(tab_hbm.at[idx_v.at[0]], buf0, s0)
        pltpu.async_copy(xyz_hbm.at[idx_v.at[0]], xb0, s0)
        pltpu.async_copy(tab_hbm.at[idx_v.at[1]], buf1, s1)
        pltpu.async_copy(xyz_hbm.at[idx_v.at[1]], xb1, s1)

        def body(gg, _):
            g0 = gg * 2
            pltpu.make_async_copy(tab_hbm.at[idx_v.at[g0]], buf0, s0).wait()
            pltpu.make_async_copy(xyz_hbm.at[idx_v.at[g0]], xb0, s0).wait()
            o = base + g0 * ch
            pltpu.sync_copy(buf0, tabg_hbm.at[pl.ds(o, ch)])
            pltpu.sync_copy(xb0, xg_hbm.at[pl.ds(o, ch)])

            @pl.when(g0 + 2 < nch)
            def _():
                pltpu.async_copy(tab_hbm.at[idx_v.at[g0 + 2]], buf0, s0)
                pltpu.async_copy(xyz_hbm.at[idx_v.at[g0 + 2]], xb0, s0)

            pltpu.make_async_copy(tab_hbm.at[idx_v.at[g0 + 1]], buf1,
                                  s1).wait()
            pltpu.make_async_copy(xyz_hbm.at[idx_v.at[g0 + 1]], xb1,
                                  s1).wait()
            o1 = o + ch
            pltpu.sync_copy(buf1, tabg_hbm.at[pl.ds(o1, ch)])
            pltpu.sync_copy(xb1, xg_hbm.at[pl.ds(o1, ch)])

            @pl.when(g0 + 3 < nch)
            def _():
                pltpu.async_copy(tab_hbm.at[idx_v.at[g0 + 3]], buf1, s1)
                pltpu.async_copy(xyz_hbm.at[idx_v.at[g0 + 3]], xb1, s1)

            return 0

        lax.fori_loop(0, nch // 2, body, 0)

    return gather


def _pointwise_body(feat_ref, makT_ref, wvT_ref, tab_ref):
    i32 = jnp.int32
    f = feat_ref[...]
    ak = jnp.dot(f, makT_ref[...], preferred_element_type=jnp.float32)
    vf = jnp.dot(f, wvT_ref[...], preferred_element_type=jnp.float32)
    akb = jax.lax.bitcast_convert_type(ak, i32) + i32(0x8000)
    vfb = jax.lax.bitcast_convert_type(vf, i32) + i32(0x8000)
    hi = akb & i32(-65536)
    lo = jax.lax.shift_right_logical(vfb, 16)
    tab_ref[...] = hi | lo


def _attn_body(feat_ref, tabg_ref, xg_ref, xq_ref, maqT_ref, p1wT_ref,
               m1T_ref, a2wT_ref, p2wT_ref, bias_ref, out_ref, *, blk):
    r = blk * KNN
    p1b = bias_ref[0:1, :]
    c1 = bias_ref[1:2, :]
    p2b = bias_ref[2:3, :]
    gamma = bias_ref[3:4, :]
    beta = bias_ref[4:5, :]

    i32 = jnp.int32
    packed = tabg_ref[...]
    akg = jax.lax.bitcast_convert_type(packed & i32(-65536), jnp.float32)
    vfg = jax.lax.bitcast_convert_type(
        jax.lax.shift_left(packed, i32(16)), jnp.float32)
    rel = (xq_ref[...][:, None, :]
           - xg_ref[...].reshape(blk, KNN, XPAD)).reshape(r, XPAD)
    u = jnp.maximum(
        jnp.dot(rel, p1wT_ref[...], preferred_element_type=jnp.float32) + p1b,
        0.0)
    t = jnp.dot(u, m1T_ref[...], preferred_element_type=jnp.float32)
    f = feat_ref[...]
    aqc = jnp.dot(f, maqT_ref[...], preferred_element_type=jnp.float32) + c1
    h = (t.reshape(blk, KNN, DIM) - akg.reshape(blk, KNN, DIM)
         + aqc[:, None, :])
    h2 = jnp.maximum(h, 0.0)
    h3 = jnp.dot(h2.reshape(r, DIM), a2wT_ref[...],
                 preferred_element_type=jnp.float32).reshape(blk, KNN, DIM)
    m = jnp.max(h3, axis=1, keepdims=True)
    e = jnp.exp(h3 - m)
    s = jnp.sum(e, axis=1, keepdims=True)
    attn = e / s
    out_v = jnp.sum(vfg.reshape(blk, KNN, DIM) * attn, axis=1)
    su = jnp.sum(u.reshape(blk, KNN, DIM) * attn, axis=1)
    y = (out_v + jnp.dot(su, p2wT_ref[...], preferred_element_type=jnp.float32)
         + p2b + f)
    mu = jnp.mean(y, axis=1, keepdims=True)
    yc = y - mu
    var = jnp.mean(yc * yc, axis=1, keepdims=True)
    out_ref[...] = yc * jax.lax.rsqrt(var + 1e-5) * gamma + beta


def kernel(xyz, feat, Wq, Wk, Wv, p1w, p1b, p2w, p2b, a1w, a1b, a2w, a2b,
           gamma, beta):
    B, N, _ = xyz.shape
    BN = B * N
    f32 = jnp.float32

    maqT = (a1w @ Wq).T
    makT = (a1w @ Wk).T
    wvT = Wv.T
    m1T = (a1w @ p2w).T
    a2wT = a2w.T
    p2wT = p2w.T
    p1wT = jnp.zeros((XPAD, DIM), f32).at[:3, :].set(p1w.T)
    c1 = a1b + a1w @ p2b
    bias_tab = jnp.zeros((8, DIM), f32)
    bias_tab = bias_tab.at[0].set(p1b).at[1].set(c1).at[2].set(p2b)
    bias_tab = bias_tab.at[3].set(gamma).at[4].set(beta)

    blkr = 64
    xyzT = jnp.zeros((B, 8, N), f32).at[:, :3, :].set(
        jnp.transpose(xyz, (0, 2, 1)))
    xb8 = jnp.zeros((B, N, 8), f32).at[..., :3].set(xyz)
    knn_idx = pl.pallas_call(
        functools.partial(_knn_body, blkr=blkr, n=N),
        grid=(B, N // blkr),
        in_specs=[
            pl.BlockSpec((1, 8, N), lambda b, i: (b, 0, 0)),
            pl.BlockSpec((1, blkr, 8), lambda b, i: (b, i, 0)),
        ],
        out_specs=pl.BlockSpec((1, blkr, KNN), lambda b, i: (b, i, 0)),
        out_shape=jax.ShapeDtypeStruct((B, N, KNN), jnp.int32),
    )(xyzT, xb8)

    feat2 = feat.reshape(BN, DIM)
    xyzP = jnp.zeros((BN, XPAD), f32).at[:, :3].set(xyz.reshape(BN, 3))
    blk_a = 512
    tab = pl.pallas_call(
        _pointwise_body,
        grid=(BN // blk_a,),
        in_specs=[
            pl.BlockSpec((blk_a, DIM), lambda i: (i, 0)),
            pl.BlockSpec((DIM, DIM), lambda i: (0, 0)),
            pl.BlockSpec((DIM, DIM), lambda i: (0, 0)),
        ],
        out_specs=pl.BlockSpec((blk_a, DIM), lambda i: (i, 0)),
        out_shape=jax.ShapeDtypeStruct((BN, DIM), jnp.int32),
    )(feat2, makT, wvT)

    nrows = BN * KNN
    gidx = (knn_idx + (jnp.arange(B, dtype=jnp.int32) * N)[:, None, None])
    gidx3 = gidx.reshape(32, nrows // (32 * 64), 64)
    tabg, xg = _make_sc_gather(nrows)(tab, xyzP, gidx3)

    blk = 64
    rblk = blk * KNN
    wspec = pl.BlockSpec((DIM, DIM), lambda i: (0, 0))
    y = pl.pallas_call(
        functools.partial(_attn_body, blk=blk),
        grid=(BN // blk,),
        in_specs=[
            pl.BlockSpec((blk, DIM), lambda i: (i, 0)),
            pl.BlockSpec((rblk, DIM), lambda i: (i, 0)),
            pl.BlockSpec((rblk, XPAD), lambda i: (i, 0)),
            pl.BlockSpec((blk, XPAD), lambda i: (i, 0)),
            wspec,
            pl.BlockSpec((XPAD, DIM), lambda i: (0, 0)),
            wspec,
            wspec,
            wspec,
            pl.BlockSpec((8, DIM), lambda i: (0, 0)),
        ],
        out_specs=pl.BlockSpec((blk, DIM), lambda i: (i, 0)),
        out_shape=jax.ShapeDtypeStruct((BN, DIM), f32),
    )(feat2, tabg, xg, xyzP, maqT, p1wT, m1T, a2wT, p2wT, bias_tab)

    return y.reshape(B, N, DIM)
